# baseline (speedup 1.0000x reference)
import functools

import jax
import jax.numpy as jnp
from jax import lax
from jax.experimental import pallas as pl
from jax.experimental.pallas import tpu as pltpu
from jax.experimental.pallas import tpu_sc as plsc

DIM = 64
MAX_LEN = 8192
BATCH = 16384
NUM_CORES = 2
NUM_SUBCORES = 16
NUM_WORKERS = NUM_CORES * NUM_SUBCORES
TILE_ROWS = DIM // 8
N_PAIRS = 4
TAB_CTILES = MAX_LEN // 128
OUT_CTILES_TOTAL = BATCH // 128
N_CHUNKS = 4
CT_PER_CHUNK = OUT_CTILES_TOTAL // N_CHUNKS
POS_PER_CHUNK = BATCH // N_CHUNKS


def _gather_body(pos_hbm, pe4_hbm, out4_hbm, idx_v, tab_v, out_v, sem, isems):
    wid = lax.axis_index("s") * NUM_CORES + lax.axis_index("c")
    k = wid // N_PAIRS
    p = wid % N_PAIRS
    cp_tab = pltpu.async_copy(
        pe4_hbm.at[k, :, pl.ds(2 * p, 2), :], tab_v, sem
    )
    idx_cps = [
        pltpu.async_copy(
            pos_hbm.at[pl.ds(c * POS_PER_CHUNK, POS_PER_CHUNK)],
            idx_v.at[pl.ds(c * POS_PER_CHUNK, POS_PER_CHUNK)],
            isems.at[c],
        )
        for c in range(N_CHUNKS)
    ]
    cp_tab.wait()

    out_cps = []
    for c in range(N_CHUNKS):
        idx_cps[c].wait()

        @plsc.parallel_loop(c * CT_PER_CHUNK, (c + 1) * CT_PER_CHUNK, 1, unroll=2)
        def ctile(ct):
            ibase = ct * 128
            for s4 in range(2):
                vs = []
                for s in range(4 * s4, 4 * s4 + 4):
                    pvec = idx_v[pl.ds(ibase + s * 16, 16)]
                    hi = lax.shift_right_logical(pvec, 7)
                    lane = lax.bitwise_and(pvec, jnp.int32(127))
                    for j in range(2):
                        jvec = jnp.full((16,), j, jnp.int32)
                        vs.append(
                            (s, j, plsc.load_gather(tab_v, [hi, jvec, lane]))
                        )
                for s, j, v in vs:
                    out_v[ct, j, pl.ds(s * 16, 16)] = v

    del out_cps
    pltpu.sync_copy(out_v, out4_hbm.at[k, :, pl.ds(2 * p, 2), :])


@jax.jit
def kernel(positions, pe):
    pe4 = pe.T.reshape(TILE_ROWS, 8, TAB_CTILES, 128).transpose(0, 2, 1, 3)
    mesh = plsc.VectorSubcoreMesh(core_axis_name="c", subcore_axis_name="s")
    run = functools.partial(
        pl.kernel,
        mesh=mesh,
        out_type=jax.ShapeDtypeStruct(
            (TILE_ROWS, OUT_CTILES_TOTAL, 8, 128), jnp.float32
        ),
        scratch_types=[
            pltpu.VMEM((BATCH,), jnp.int32),
            pltpu.VMEM((TAB_CTILES, 2, 128), jnp.float32),
            pltpu.VMEM((OUT_CTILES_TOTAL, 2, 128), jnp.float32),
            pltpu.SemaphoreType.DMA,
            pltpu.SemaphoreType.DMA((N_CHUNKS,)),
        ],
        compiler_params=pltpu.CompilerParams(
            use_tc_tiling_on_sc=False, needs_layout_passes=False
        ),
    )(_gather_body)
    out4 = run(positions.astype(jnp.int32), pe4)
    return out4.transpose(0, 2, 1, 3).reshape(DIM, BATCH).T


# speedup vs baseline: 1.0244x; 1.0244x over previous
import functools

import jax
import jax.numpy as jnp
from jax import lax
from jax.experimental import pallas as pl
from jax.experimental.pallas import tpu as pltpu
from jax.experimental.pallas import tpu_sc as plsc

DIM = 64
MAX_LEN = 8192
BATCH = 16384
NUM_CORES = 2
NUM_SUBCORES = 16
NUM_WORKERS = NUM_CORES * NUM_SUBCORES
TILE_ROWS = DIM // 8
N_PAIRS = 4
TAB_CTILES = MAX_LEN // 128
OUT_CTILES_TOTAL = BATCH // 128


def _gather_body(pos_hbm, pe4_hbm, out4_hbm, idx_v, tab_v, out_v, sem):
    wid = lax.axis_index("s") * NUM_CORES + lax.axis_index("c")
    k = wid // N_PAIRS
    p = wid % N_PAIRS
    cp_tab = pltpu.async_copy(
        pe4_hbm.at[k, :, pl.ds(2 * p, 2), :], tab_v, sem
    )
    cp_idx = pltpu.async_copy(pos_hbm, idx_v, sem)
    cp_tab.wait()
    cp_idx.wait()

    @plsc.parallel_loop(0, OUT_CTILES_TOTAL, 1, unroll=2)
    def ctile(ct):
        ibase = ct * 128
        for s4 in range(2):
            vs = []
            for s in range(4 * s4, 4 * s4 + 4):
                pvec = idx_v[pl.ds(ibase + s * 16, 16)]
                hi = lax.shift_right_logical(pvec, 7)
                lane = lax.bitwise_and(pvec, jnp.int32(127))
                for j in range(2):
                    jvec = jnp.full((16,), j, jnp.int32)
                    vs.append((s, j, plsc.load_gather(tab_v, [hi, jvec, lane])))
            for s, j, v in vs:
                out_v[ct, j, pl.ds(s * 16, 16)] = v

    pltpu.sync_copy(out_v, out4_hbm.at[k, :, pl.ds(2 * p, 2), :])


@jax.jit
def kernel(positions, pe):
    pe4 = pe.T.reshape(TILE_ROWS, 8, TAB_CTILES, 128).transpose(0, 2, 1, 3)
    mesh = plsc.VectorSubcoreMesh(core_axis_name="c", subcore_axis_name="s")
    run = functools.partial(
        pl.kernel,
        mesh=mesh,
        out_type=jax.ShapeDtypeStruct(
            (TILE_ROWS, OUT_CTILES_TOTAL, 8, 128), jnp.float32
        ),
        scratch_types=[
            pltpu.VMEM((BATCH,), jnp.int32),
            pltpu.VMEM((TAB_CTILES, 2, 128), jnp.float32),
            pltpu.VMEM((OUT_CTILES_TOTAL, 2, 128), jnp.float32),
            pltpu.SemaphoreType.DMA,
        ],
        compiler_params=pltpu.CompilerParams(
            use_tc_tiling_on_sc=False, needs_layout_passes=False
        ),
    )(_gather_body)
    out4 = run(positions.astype(jnp.int32), pe4)
    return out4.transpose(0, 2, 1, 3).reshape(DIM, BATCH).T


# speedup vs baseline: 1.1328x; 1.1058x over previous
import functools

import jax
import jax.numpy as jnp
from jax import lax
from jax.experimental import pallas as pl
from jax.experimental.pallas import tpu as pltpu
from jax.experimental.pallas import tpu_sc as plsc

DIM = 64
MAX_LEN = 8192
BATCH = 16384
NUM_CORES = 2
NUM_SUBCORES = 16
NUM_WORKERS = NUM_CORES * NUM_SUBCORES
TILE_ROWS = DIM // 8
N_PAIRS = 4
TAB_CTILES = MAX_LEN // 128
OUT_CTILES_TOTAL = BATCH // 128


def _gather_body(pos_hbm, pe4_hbm, out4_hbm, idx_v, tab_v, out_v, idx_sh, sem):
    sid = lax.axis_index("s")
    wid = sid * NUM_CORES + lax.axis_index("c")
    k = wid // N_PAIRS
    p = wid % N_PAIRS
    cp_tab = pltpu.async_copy(
        pe4_hbm.at[k, :, pl.ds(2 * p, 2), :], tab_v, sem
    )
    @pl.when(sid == 0)
    def _():
        pltpu.sync_copy(pos_hbm, idx_sh)

    plsc.subcore_barrier()
    cp_idx = pltpu.async_copy(idx_sh, idx_v, sem)
    cp_tab.wait()
    cp_idx.wait()

    @plsc.parallel_loop(0, OUT_CTILES_TOTAL, 1, unroll=2)
    def ctile(ct):
        ibase = ct * 128
        for s4 in range(2):
            vs = []
            for s in range(4 * s4, 4 * s4 + 4):
                pvec = idx_v[pl.ds(ibase + s * 16, 16)]
                hi = lax.shift_right_logical(pvec, 7)
                lane = lax.bitwise_and(pvec, jnp.int32(127))
                for j in range(2):
                    jvec = jnp.full((16,), j, jnp.int32)
                    vs.append((s, j, plsc.load_gather(tab_v, [hi, jvec, lane])))
            for s, j, v in vs:
                out_v[ct, j, pl.ds(s * 16, 16)] = v

    pltpu.sync_copy(out_v, out4_hbm.at[k, :, pl.ds(2 * p, 2), :])


@jax.jit
def kernel(positions, pe):
    pe4 = pe.T.reshape(TILE_ROWS, 8, TAB_CTILES, 128).transpose(0, 2, 1, 3)
    mesh = plsc.VectorSubcoreMesh(core_axis_name="c", subcore_axis_name="s")
    run = functools.partial(
        pl.kernel,
        mesh=mesh,
        out_type=jax.ShapeDtypeStruct(
            (TILE_ROWS, OUT_CTILES_TOTAL, 8, 128), jnp.float32
        ),
        scratch_types=[
            pltpu.VMEM((BATCH,), jnp.int32),
            pltpu.VMEM((TAB_CTILES, 2, 128), jnp.float32),
            pltpu.VMEM((OUT_CTILES_TOTAL, 2, 128), jnp.float32),
            pltpu.VMEM_SHARED((BATCH,), jnp.int32),
            pltpu.SemaphoreType.DMA,
        ],
        compiler_params=pltpu.CompilerParams(
            use_tc_tiling_on_sc=False, needs_layout_passes=False
        ),
    )(_gather_body)
    out4 = run(positions.astype(jnp.int32), pe4)
    return out4.transpose(0, 2, 1, 3).reshape(DIM, BATCH).T


# speedup vs baseline: 1.1504x; 1.0155x over previous
import functools

import jax
import jax.numpy as jnp
from jax import lax
from jax.experimental import pallas as pl
from jax.experimental.pallas import tpu as pltpu
from jax.experimental.pallas import tpu_sc as plsc

DIM = 64
MAX_LEN = 8192
BATCH = 16384
NUM_CORES = 2
NUM_SUBCORES = 16
NUM_WORKERS = NUM_CORES * NUM_SUBCORES
TILE_ROWS = DIM // 8
N_PAIRS = 4
TAB_CTILES = MAX_LEN // 128
OUT_CTILES_TOTAL = BATCH // 128


def _gather_body(pos_hbm, pe4_hbm, out4_hbm, idx_v, tab_v, out_v, idx_sh, sem):
    sid = lax.axis_index("s")
    wid = sid * NUM_CORES + lax.axis_index("c")
    k = wid // N_PAIRS
    p = wid % N_PAIRS
    cp_tab = pltpu.async_copy(
        pe4_hbm.at[k, :, pl.ds(2 * p, 2), :], tab_v, sem
    )
    @pl.when(sid == 0)
    def _():
        pltpu.sync_copy(pos_hbm, idx_sh)

    plsc.subcore_barrier()
    cp_idx = pltpu.async_copy(idx_sh, idx_v, sem)
    cp_tab.wait()
    cp_idx.wait()

    half = OUT_CTILES_TOTAL // 2
    out_cps = []
    for h in range(2):

        @plsc.parallel_loop(h * half, (h + 1) * half, 1, unroll=2)
        def ctile(ct):
            ibase = ct * 128
            for s4 in range(2):
                vs = []
                for s in range(4 * s4, 4 * s4 + 4):
                    pvec = idx_v[pl.ds(ibase + s * 16, 16)]
                    hi = lax.shift_right_logical(pvec, 7)
                    lane = lax.bitwise_and(pvec, jnp.int32(127))
                    for j in range(2):
                        jvec = jnp.full((16,), j, jnp.int32)
                        vs.append(
                            (s, j, plsc.load_gather(tab_v, [hi, jvec, lane]))
                        )
                for s, j, v in vs:
                    out_v[ct, j, pl.ds(s * 16, 16)] = v

        out_cps.append(
            pltpu.async_copy(
                out_v.at[pl.ds(h * half, half)],
                out4_hbm.at[k, pl.ds(h * half, half), pl.ds(2 * p, 2), :],
                sem,
            )
        )
    for cp in out_cps:
        cp.wait()


@jax.jit
def kernel(positions, pe):
    pe4 = pe.T.reshape(TILE_ROWS, 8, TAB_CTILES, 128).transpose(0, 2, 1, 3)
    mesh = plsc.VectorSubcoreMesh(core_axis_name="c", subcore_axis_name="s")
    run = functools.partial(
        pl.kernel,
        mesh=mesh,
        out_type=jax.ShapeDtypeStruct(
            (TILE_ROWS, OUT_CTILES_TOTAL, 8, 128), jnp.float32
        ),
        scratch_types=[
            pltpu.VMEM((BATCH,), jnp.int32),
            pltpu.VMEM((TAB_CTILES, 2, 128), jnp.float32),
            pltpu.VMEM((OUT_CTILES_TOTAL, 2, 128), jnp.float32),
            pltpu.VMEM_SHARED((BATCH,), jnp.int32),
            pltpu.SemaphoreType.DMA,
        ],
        compiler_params=pltpu.CompilerParams(
            use_tc_tiling_on_sc=False, needs_layout_passes=False
        ),
    )(_gather_body)
    out4 = run(positions.astype(jnp.int32), pe4)
    return out4.transpose(0, 2, 1, 3).reshape(DIM, BATCH).T
